# baseline (speedup 1.0000x reference)
import functools
import jax
import jax.numpy as jnp
from jax import lax
from jax.experimental import pallas as pl
from jax.experimental.pallas import tpu as pltpu
from jax.experimental.pallas import tpu_sc as plsc

_B, _V = 16384, 130
_F = _V - 2
_COUNT = 3 * _F * _B
_NW = 32
_LPW = _B // _NW
_CH = 128
_NCH = _LPW // _CH
_VP = 136
_MAGIC = jnp.int32(0x5F3759DF)

_sc_mesh = plsc.VectorSubcoreMesh(core_axis_name="c", subcore_axis_name="s")


def _sqrt16(z):
    z = jnp.maximum(z, jnp.float32(1e-12))
    y = plsc.bitcast(_MAGIC - (plsc.bitcast(z, jnp.int32) >> 1), jnp.float32)
    hz = jnp.float32(0.5) * z
    y = y * (jnp.float32(1.5) - hz * y * y)
    y = y * (jnp.float32(1.5) - hz * y * y)
    return z * y


@functools.partial(
    pl.kernel,
    mesh=_sc_mesh,
    out_type=jax.ShapeDtypeStruct((_NW * 8, 16), jnp.float32),
    scratch_types=[
        pltpu.VMEM((3 * _VP, _CH), jnp.float32),
        pltpu.VMEM((3 * _VP, _CH), jnp.float32),
        pltpu.VMEM((8, 16), jnp.float32),
    ],
    compiler_params=pltpu.CompilerParams(needs_layout_passes=False),
)
def _sc_loss(xo_hbm, xg_hbm, out_hbm, xov, xgv, ov):
    wid = lax.axis_index("s") * 2 + lax.axis_index("c")
    acc = jnp.zeros((16,), jnp.float32)
    lane = lax.iota(jnp.int32, 16)
    cols = [lane + l * 16 for l in range(_CH // 16)]

    def dists(xv, rows, col):
        n2 = jnp.float32(0)
        s2 = jnp.float32(0)
        for t in range(3):
            x0 = plsc.load_gather(xv, [rows[t], col])
            x1 = plsc.load_gather(xv, [rows[t] + 1, col])
            x2 = plsc.load_gather(xv, [rows[t] + 2, col])
            a = x1 - x0
            b = x2 - x0
            n2 = n2 + a * a
            s2 = s2 + b * b
        return _sqrt16(n2), _sqrt16(s2)

    def last_dist(xv, rows, col):
        n2 = jnp.float32(0)
        for t in range(3):
            x0 = plsc.load_gather(xv, [rows[t] + (_V - 2), col])
            x1 = plsc.load_gather(xv, [rows[t] + (_V - 1), col])
            a = x1 - x0
            n2 = n2 + a * a
        return _sqrt16(n2)

    zrows = [jnp.zeros((16,), jnp.int32) + t * _VP for t in range(3)]

    for c in range(_NCH):
        base = pl.multiple_of(wid * _LPW + c * _CH, _CH)
        for t in range(3):
            pltpu.sync_copy(xo_hbm.at[t, :, pl.ds(base, _CH)],
                            xov.at[pl.ds(t * _VP, _V), :])
            pltpu.sync_copy(xg_hbm.at[t, :, pl.ds(base, _CH)],
                            xgv.at[pl.ds(t * _VP, _V), :])

        def vbody(v, a_in):
            a = a_in
            wv = jnp.where(v == 0, jnp.float32(1.0), jnp.float32(2.0))
            rows = [zr + v for zr in zrows]
            for l in range(_CH // 16):
                no, so = dists(xov, rows, cols[l])
                ng, sg = dists(xgv, rows, cols[l])
                a = a + (wv * jnp.abs(no - ng) + jnp.abs(so - sg))
            return a

        acc = lax.fori_loop(0, _V - 2, vbody, acc)
        for l in range(_CH // 16):
            no = last_dist(xov, zrows, cols[l])
            ng = last_dist(xgv, zrows, cols[l])
            acc = acc + jnp.abs(no - ng)

    zero16 = jnp.zeros((16,), jnp.float32)
    for r in range(1, 8):
        ov[r, :] = zero16
    ov[0, :] = acc
    pltpu.sync_copy(ov, out_hbm.at[pl.ds(wid * 8, 8)])


@jax.jit
def kernel(coord_out, coord_gt):
    xo = coord_out.transpose(2, 1, 0)
    xg = coord_gt.transpose(2, 1, 0)
    parts = _sc_loss(xo, xg)
    return jnp.sum(parts) * (1.0 / _COUNT)


# speedup vs baseline: 2.5319x; 2.5319x over previous
import functools
import jax
import jax.numpy as jnp
from jax import lax
from jax.experimental import pallas as pl
from jax.experimental.pallas import tpu as pltpu
from jax.experimental.pallas import tpu_sc as plsc

_B, _V = 16384, 130
_F = _V - 2
_COUNT = 3 * _F * _B

_SPLIT = 4096
_NW = 32
_LPW = _SPLIT // _NW
_CH = 128
_NCH = _LPW // _CH
_VP = 136
_MAGIC = jnp.int32(0x5F3759DF)

_sc_mesh = plsc.VectorSubcoreMesh(core_axis_name="c", subcore_axis_name="s")


def _sqrt16(z):
    z = jnp.maximum(z, jnp.float32(1e-12))
    y = plsc.bitcast(_MAGIC - (plsc.bitcast(z, jnp.int32) >> 1), jnp.float32)
    hz = jnp.float32(0.5) * z
    y = y * (jnp.float32(1.5) - hz * y * y)
    y = y * (jnp.float32(1.5) - hz * y * y)
    return z * y


@functools.partial(
    pl.kernel,
    mesh=_sc_mesh,
    out_type=jax.ShapeDtypeStruct((_NW * 8, 16), jnp.float32),
    scratch_types=[
        pltpu.VMEM((3 * _VP, _CH), jnp.float32),
        pltpu.VMEM((3 * _VP, _CH), jnp.float32),
        pltpu.VMEM((8, 16), jnp.float32),
    ],
    compiler_params=pltpu.CompilerParams(needs_layout_passes=False),
)
def _sc_loss(xo_hbm, xg_hbm, out_hbm, xov, xgv, ov):
    wid = lax.axis_index("s") * 2 + lax.axis_index("c")
    acc = jnp.zeros((16,), jnp.float32)
    lane = lax.iota(jnp.int32, 16)
    cols = [lane + l * 16 for l in range(_CH // 16)]

    def dists(xv, rows, col):
        n2 = jnp.float32(0)
        s2 = jnp.float32(0)
        for t in range(3):
            x0 = plsc.load_gather(xv, [rows[t], col])
            x1 = plsc.load_gather(xv, [rows[t] + 1, col])
            x2 = plsc.load_gather(xv, [rows[t] + 2, col])
            a = x1 - x0
            b = x2 - x0
            n2 = n2 + a * a
            s2 = s2 + b * b
        return _sqrt16(n2), _sqrt16(s2)

    def last_dist(xv, rows, col):
        n2 = jnp.float32(0)
        for t in range(3):
            x0 = plsc.load_gather(xv, [rows[t] + (_V - 2), col])
            x1 = plsc.load_gather(xv, [rows[t] + (_V - 1), col])
            a = x1 - x0
            n2 = n2 + a * a
        return _sqrt16(n2)

    zrows = [jnp.zeros((16,), jnp.int32) + t * _VP for t in range(3)]

    for c in range(_NCH):
        base = pl.multiple_of(wid * _LPW + c * _CH, _CH)
        for t in range(3):
            pltpu.sync_copy(xo_hbm.at[t, :, pl.ds(base, _CH)],
                            xov.at[pl.ds(t * _VP, _V), :])
            pltpu.sync_copy(xg_hbm.at[t, :, pl.ds(base, _CH)],
                            xgv.at[pl.ds(t * _VP, _V), :])

        def vbody(v, a_in):
            a = a_in
            wv = jnp.where(v == 0, jnp.float32(1.0), jnp.float32(2.0))
            rows = [zr + v for zr in zrows]
            for l in range(_CH // 16):
                no, so = dists(xov, rows, cols[l])
                ng, sg = dists(xgv, rows, cols[l])
                a = a + (wv * jnp.abs(no - ng) + jnp.abs(so - sg))
            return a

        acc = lax.fori_loop(0, _V - 2, vbody, acc)
        for l in range(_CH // 16):
            no = last_dist(xov, zrows, cols[l])
            ng = last_dist(xgv, zrows, cols[l])
            acc = acc + jnp.abs(no - ng)

    zero16 = jnp.zeros((16,), jnp.float32)
    for r in range(1, 8):
        ov[r, :] = zero16
    ov[0, :] = acc
    pltpu.sync_copy(ov, out_hbm.at[pl.ds(wid * 8, 8)])


_BL = 2048
_TC_GRID = (_B - _SPLIT) // _BL


def _tc_body(xo_ref, xg_ref, o_ref):
    pid = pl.program_id(0)

    @pl.when(pid == 0)
    def _():
        o_ref[0, 0] = 0.0

    def edge_dists(x):
        e = x[:, 1:, :] - x[:, :-1, :]
        e2 = e * e
        n2 = e2[0] + e2[1] + e2[2]
        f = x[:, 2:, :] - x[:, :-2, :]
        f2 = f * f
        s2 = f2[0] + f2[1] + f2[2]
        return jnp.sqrt(n2), jnp.sqrt(s2)

    no, so = edge_dists(xo_ref[...])
    ng, sg = edge_dists(xg_ref[...])
    dn = jnp.abs(no - ng)
    ds = jnp.abs(so - sg)
    part = (2.0 * jnp.sum(dn) - jnp.sum(dn[0]) - jnp.sum(dn[128])
            + jnp.sum(ds))
    o_ref[0, 0] += part


@jax.jit
def kernel(coord_out, coord_gt):
    xo = coord_out.transpose(2, 1, 0)
    xg = coord_gt.transpose(2, 1, 0)
    sc_parts = _sc_loss(xo, xg)
    tc_acc = pl.pallas_call(
        _tc_body,
        grid=(_TC_GRID,),
        in_specs=[
            pl.BlockSpec((3, _V, _BL), lambda i: (0, 0, i + _SPLIT // _BL)),
            pl.BlockSpec((3, _V, _BL), lambda i: (0, 0, i + _SPLIT // _BL)),
        ],
        out_specs=pl.BlockSpec(memory_space=pltpu.SMEM),
        out_shape=jax.ShapeDtypeStruct((1, 1), jnp.float32),
        compiler_params=pltpu.CompilerParams(
            dimension_semantics=("arbitrary",)),
    )(xo, xg)
    return (jnp.sum(sc_parts) + tc_acc[0, 0]) * (1.0 / _COUNT)


# speedup vs baseline: 2.6466x; 1.0453x over previous
import functools
import jax
import jax.numpy as jnp
from jax import lax
from jax.experimental import pallas as pl
from jax.experimental.pallas import tpu as pltpu
from jax.experimental.pallas import tpu_sc as plsc

_B, _V = 16384, 130
_F = _V - 2
_COUNT = 3 * _F * _B

_SPLIT = 4096
_NW = 32
_LPW = _SPLIT // _NW
_CH = 128
_NCH = _LPW // _CH
_VP = 136
_MAGIC = jnp.int32(0x5F3759DF)

_sc_mesh = plsc.VectorSubcoreMesh(core_axis_name="c", subcore_axis_name="s")


def _sqrt16(z):
    z = jnp.maximum(z, jnp.float32(1e-12))
    y = plsc.bitcast(_MAGIC - (plsc.bitcast(z, jnp.int32) >> 1), jnp.float32)
    hz = jnp.float32(0.5) * z
    y = y * (jnp.float32(1.5) - hz * y * y)
    y = y * (jnp.float32(1.5) - hz * y * y)
    return z * y


@functools.partial(
    pl.kernel,
    mesh=_sc_mesh,
    out_type=jax.ShapeDtypeStruct((_NW * 8, 16), jnp.float32),
    scratch_types=[
        pltpu.VMEM((3 * _VP, _CH), jnp.float32),
        pltpu.VMEM((3 * _VP, _CH), jnp.float32),
        pltpu.VMEM((8, 16), jnp.float32),
        pltpu.SemaphoreType.DMA,
    ],
    compiler_params=pltpu.CompilerParams(needs_layout_passes=False),
)
def _sc_loss(xo_hbm, xg_hbm, out_hbm, xov, xgv, ov, dsem):
    wid = lax.axis_index("s") * 2 + lax.axis_index("c")
    acc = jnp.zeros((16,), jnp.float32)
    lane = lax.iota(jnp.int32, 16)
    cols = [lane + l * 16 for l in range(_CH // 16)]

    def dists(xv, rows, col):
        n2 = jnp.float32(0)
        s2 = jnp.float32(0)
        for t in range(3):
            x0 = plsc.load_gather(xv, [rows[t], col])
            x1 = plsc.load_gather(xv, [rows[t] + 1, col])
            x2 = plsc.load_gather(xv, [rows[t] + 2, col])
            a = x1 - x0
            b = x2 - x0
            n2 = n2 + a * a
            s2 = s2 + b * b
        return _sqrt16(n2), _sqrt16(s2)

    def last_dist(xv, rows, col):
        n2 = jnp.float32(0)
        for t in range(3):
            x0 = plsc.load_gather(xv, [rows[t] + (_V - 2), col])
            x1 = plsc.load_gather(xv, [rows[t] + (_V - 1), col])
            a = x1 - x0
            n2 = n2 + a * a
        return _sqrt16(n2)

    zrows = [jnp.zeros((16,), jnp.int32) + t * _VP for t in range(3)]

    for c in range(_NCH):
        base = pl.multiple_of(wid * _LPW + c * _CH, _CH)
        cps = []
        for t in range(3):
            cps.append(pltpu.async_copy(xo_hbm.at[t, :, pl.ds(base, _CH)],
                                        xov.at[pl.ds(t * _VP, _V), :], dsem))
            cps.append(pltpu.async_copy(xg_hbm.at[t, :, pl.ds(base, _CH)],
                                        xgv.at[pl.ds(t * _VP, _V), :], dsem))
        for cp in cps:
            cp.wait()

        def vbody(v, a_in):
            a = a_in
            wv = jnp.where(v == 0, jnp.float32(1.0), jnp.float32(2.0))
            rows = [zr + v for zr in zrows]
            for l in range(_CH // 16):
                no, so = dists(xov, rows, cols[l])
                ng, sg = dists(xgv, rows, cols[l])
                a = a + (wv * jnp.abs(no - ng) + jnp.abs(so - sg))
            return a

        acc = lax.fori_loop(0, _V - 2, vbody, acc)
        for l in range(_CH // 16):
            no = last_dist(xov, zrows, cols[l])
            ng = last_dist(xgv, zrows, cols[l])
            acc = acc + jnp.abs(no - ng)

    zero16 = jnp.zeros((16,), jnp.float32)
    for r in range(1, 8):
        ov[r, :] = zero16
    ov[0, :] = acc
    pltpu.sync_copy(ov, out_hbm.at[pl.ds(wid * 8, 8)])


_BL = 2048
_TC_GRID = (_B - _SPLIT) // _BL


def _tc_body(xo_ref, xg_ref, o_ref):
    pid = pl.program_id(0)

    @pl.when(pid == 0)
    def _():
        o_ref[0, 0] = 0.0

    def edge_dists(x):
        e = x[:, 1:, :] - x[:, :-1, :]
        e2 = e * e
        n2 = e2[0] + e2[1] + e2[2]
        f = x[:, 2:, :] - x[:, :-2, :]
        f2 = f * f
        s2 = f2[0] + f2[1] + f2[2]
        return jnp.sqrt(n2), jnp.sqrt(s2)

    no, so = edge_dists(xo_ref[...])
    ng, sg = edge_dists(xg_ref[...])
    dn = jnp.abs(no - ng)
    ds = jnp.abs(so - sg)
    part = (2.0 * jnp.sum(dn) - jnp.sum(dn[0]) - jnp.sum(dn[128])
            + jnp.sum(ds))
    o_ref[0, 0] += part


@jax.jit
def kernel(coord_out, coord_gt):
    xo = coord_out.transpose(2, 1, 0)
    xg = coord_gt.transpose(2, 1, 0)
    sc_parts = _sc_loss(xo, xg)
    tc_acc = pl.pallas_call(
        _tc_body,
        grid=(_TC_GRID,),
        in_specs=[
            pl.BlockSpec((3, _V, _BL), lambda i: (0, 0, i + _SPLIT // _BL)),
            pl.BlockSpec((3, _V, _BL), lambda i: (0, 0, i + _SPLIT // _BL)),
        ],
        out_specs=pl.BlockSpec(memory_space=pltpu.SMEM),
        out_shape=jax.ShapeDtypeStruct((1, 1), jnp.float32),
        compiler_params=pltpu.CompilerParams(
            dimension_semantics=("arbitrary",)),
    )(xo, xg)
    return (jnp.sum(sc_parts) + tc_acc[0, 0]) * (1.0 / _COUNT)
